# Initial kernel scaffold; baseline (speedup 1.0000x reference)
import jax
import jax.numpy as jnp
from jax.experimental import pallas as pl


def kernel(x, weight):
    raise NotImplementedError("write your pallas kernel here")


# speedup vs baseline: 1.8445x; 1.8445x over previous
import functools

import jax
import jax.numpy as jnp
from jax import lax
from jax.experimental import pallas as pl
from jax.experimental.pallas import tpu as pltpu
from jax.experimental.pallas import tpu_sc as plsc

VOCAB = 1000000
D_MODEL = 64
B = 16384
L = 50
N = B * L

_info = plsc.get_sparse_core_info()
NC = _info.num_cores
NS = _info.num_subcores
NW = NC * NS
PER_W = N // NW

IROW = 128
G = 8
CH = G * IROW
NCHUNK = PER_W // CH

_mesh = plsc.VectorSubcoreMesh(core_axis_name="c", subcore_axis_name="s")


@functools.partial(
    pl.kernel,
    mesh=_mesh,
    out_type=jax.ShapeDtypeStruct((N, D_MODEL), jnp.float32),
    scratch_types=[
        pltpu.VMEM((G, IROW), jnp.int32),
        pltpu.VMEM((CH, D_MODEL), jnp.float32),
        pltpu.SemaphoreType.DMA,
    ],
    compiler_params=pltpu.CompilerParams(use_tc_tiling_on_sc=False),
)
def _sc_gather(idx_hbm, table_hbm, out_hbm, idx_v, rows_v, sem):
    wid = lax.axis_index("s") * NC + lax.axis_index("c")
    base = wid * PER_W

    def chunk_body(i, _):
        row_off = pl.multiple_of(base + i * CH, CH)
        pltpu.sync_copy(idx_hbm.at[pl.ds(pl.multiple_of(row_off // IROW, G), G)], idx_v)
        copies = [
            pltpu.async_copy(
                table_hbm.at[idx_v.at[j]],
                rows_v.at[pl.ds(j * IROW, IROW)],
                sem,
            )
            for j in range(G)
        ]
        for c in copies:
            c.wait()
        pltpu.sync_copy(rows_v, out_hbm.at[pl.ds(row_off, CH)])
        return _

    lax.fori_loop(0, NCHUNK, chunk_body, 0)


def kernel(x, weight):
    idx2d = x.reshape(N // IROW, IROW).astype(jnp.int32)
    out = _sc_gather(idx2d, weight)
    return out.reshape(B, L, D_MODEL)


# speedup vs baseline: 1.8761x; 1.0172x over previous
import functools

import jax
import jax.numpy as jnp
from jax import lax
from jax.experimental import pallas as pl
from jax.experimental.pallas import tpu as pltpu
from jax.experimental.pallas import tpu_sc as plsc

VOCAB = 1000000
D_MODEL = 64
B = 16384
L = 50
N = B * L

_info = plsc.get_sparse_core_info()
NC = _info.num_cores
NS = _info.num_subcores
NW = NC * NS
PER_W = N // NW

IROW = 128
G = 4
CH = G * IROW
NCHUNK = PER_W // CH
NBUF = 2
NPAIR = NCHUNK // NBUF
IDX_ROWS = PER_W // IROW

_mesh = plsc.VectorSubcoreMesh(core_axis_name="c", subcore_axis_name="s")


@functools.partial(
    pl.kernel,
    mesh=_mesh,
    out_type=jax.ShapeDtypeStruct((N, D_MODEL), jnp.float32),
    scratch_types=[
        pltpu.VMEM((IDX_ROWS, IROW), jnp.int32),
        pltpu.VMEM((NBUF, CH, D_MODEL), jnp.float32),
        pltpu.SemaphoreType.DMA,
        pltpu.SemaphoreType.DMA,
        pltpu.SemaphoreType.DMA,
        pltpu.SemaphoreType.DMA,
    ],
    compiler_params=pltpu.CompilerParams(use_tc_tiling_on_sc=False),
)
def _sc_gather(idx_hbm, table_hbm, out_hbm, idx_v, rows_v, g0, g1, o0, o1):
    gsem = (g0, g1)
    osem = (o0, o1)
    wid = lax.axis_index("s") * NC + lax.axis_index("c")
    base = wid * PER_W

    pltpu.sync_copy(
        idx_hbm.at[pl.ds(pl.multiple_of(wid * IDX_ROWS, 8), IDX_ROWS)], idx_v
    )

    def fire_gathers(i, b):
        for j in range(G):
            pltpu.async_copy(
                table_hbm.at[idx_v.at[i * G + j]],
                rows_v.at[b].at[pl.ds(j * IROW, IROW)],
                gsem[b],
            )

    def wait_gathers(b):
        pltpu.make_async_copy(
            out_hbm.at[pl.ds(0, CH)], rows_v.at[b], gsem[b]
        ).wait()

    def fire_out(i, b):
        off = pl.multiple_of(base + i * CH, CH)
        pltpu.async_copy(rows_v.at[b], out_hbm.at[pl.ds(off, CH)], osem[b])

    def wait_out(b):
        pltpu.make_async_copy(
            out_hbm.at[pl.ds(0, CH)], rows_v.at[b], osem[b]
        ).wait()

    for b in range(NBUF):
        fire_gathers(b, b)

    def body(it, carry):
        for b in range(NBUF):
            i = it * NBUF + b
            wait_gathers(b)
            fire_out(i, b)
            nxt = i + NBUF

            @pl.when(nxt < NCHUNK)
            def _():
                wait_out(b)
                fire_gathers(nxt, b)

        return carry

    lax.fori_loop(0, NPAIR, body, 0)
    for b in range(NBUF):
        wait_out(b)


def kernel(x, weight):
    idx2d = x.reshape(N // IROW, IROW).astype(jnp.int32)
    out = _sc_gather(idx2d, weight)
    return out.reshape(B, L, D_MODEL)
